# Initial kernel scaffold; baseline (speedup 1.0000x reference)
import jax
import jax.numpy as jnp
from jax.experimental import pallas as pl


def kernel(idx, val, fc_weight, bias):
    raise NotImplementedError("write your pallas kernel here")


# speedup vs baseline: 1.2065x; 1.2065x over previous
import functools

import jax
import jax.numpy as jnp
from jax import lax
from jax.experimental import pallas as pl
from jax.experimental.pallas import tpu as pltpu
from jax.experimental.pallas import tpu_sc as plsc

BATCH = 16384
FIELDS = 26
NC = 2
NS = 16
NW = NC * NS
BPW = BATCH // NW
NPW = BPW * FIELDS
LANES = 16

_mesh = plsc.VectorSubcoreMesh(core_axis_name="c", subcore_axis_name="s")


@functools.partial(
    pl.kernel,
    mesh=_mesh,
    out_type=jax.ShapeDtypeStruct((BATCH,), jnp.float32),
    scratch_types=[
        pltpu.VMEM((NPW,), jnp.int32),
        pltpu.VMEM((NPW,), jnp.float32),
        pltpu.VMEM((NPW,), jnp.float32),
        pltpu.VMEM((BPW,), jnp.float32),
        pltpu.VMEM((LANES,), jnp.float32),
        pltpu.SemaphoreType.DMA,
    ],
    compiler_params=pltpu.CompilerParams(needs_layout_passes=False),
)
def _fl_kernel(idx_hbm, val_hbm, w_hbm, b_hbm, out_hbm,
               idx_v, val_v, emb_v, out_v, bias_v, sem):
    wid = lax.axis_index("s") * NC + lax.axis_index("c")
    base = wid * NPW

    pltpu.sync_copy(idx_hbm.at[pl.ds(base, NPW)], idx_v)
    pltpu.sync_copy(val_hbm.at[pl.ds(base, NPW)], val_v)
    pltpu.sync_copy(b_hbm, bias_v.at[pl.ds(0, 1)])
    pltpu.async_copy(w_hbm.at[idx_v], emb_v, sem).wait()

    row_stride = lax.iota(jnp.int32, LANES) * FIELDS
    bias_s = bias_v[...][0]

    def chunk_body(c, carry):
        ids0 = c * (LANES * FIELDS) + row_stride
        acc = jnp.zeros((LANES,), jnp.float32)
        for f in range(FIELDS):
            ids = ids0 + f
            e = plsc.load_gather(emb_v, [ids])
            v = plsc.load_gather(val_v, [ids])
            acc = acc + e * v
        out_v[pl.ds(c * LANES, LANES)] = acc + bias_s
        return carry

    lax.fori_loop(0, BPW // LANES, chunk_body, 0)

    pltpu.sync_copy(out_v, out_hbm.at[pl.ds(wid * BPW, BPW)])


def kernel(idx, val, fc_weight, bias):
    out = _fl_kernel(idx.reshape(-1), val.reshape(-1),
                     fc_weight.reshape(-1), bias)
    return out.reshape(BATCH, 1)


# speedup vs baseline: 1.2072x; 1.0005x over previous
import functools

import jax
import jax.numpy as jnp
from jax import lax
from jax.experimental import pallas as pl
from jax.experimental.pallas import tpu as pltpu
from jax.experimental.pallas import tpu_sc as plsc

BATCH = 16384
FIELDS = 26
IN_DIM = 1000000
NC = 2
NS = 16
NW = NC * NS
BPW = BATCH // NW
NPW = BPW * FIELDS
LANES = 16

_mesh = plsc.VectorSubcoreMesh(core_axis_name="c", subcore_axis_name="s")


@functools.partial(
    pl.kernel,
    mesh=_mesh,
    out_type=jax.ShapeDtypeStruct((BATCH,), jnp.float32),
    scratch_types=[
        pltpu.VMEM((NPW,), jnp.int32),
        pltpu.VMEM((NPW,), jnp.float32),
        pltpu.VMEM((NPW,), jnp.float32),
        pltpu.VMEM((BPW,), jnp.float32),
        pltpu.VMEM((LANES,), jnp.float32),
        pltpu.SemaphoreType.DMA,
    ],
    compiler_params=pltpu.CompilerParams(
        needs_layout_passes=False, use_tc_tiling_on_sc=False),
)
def _fl_kernel(idx_hbm, val_hbm, w_hbm, b_hbm, out_hbm,
               idx_v, val_v, emb_v, out_v, bias_v, sem):
    wid = lax.axis_index("s") * NC + lax.axis_index("c")
    base = wid * NPW

    pltpu.sync_copy(idx_hbm.at[pl.ds(base, NPW)], idx_v)
    pltpu.sync_copy(val_hbm.at[pl.ds(base, NPW)], val_v)
    pltpu.sync_copy(b_hbm, bias_v.at[pl.ds(0, 1)])
    pltpu.async_copy(w_hbm.at[idx_v], emb_v, sem).wait()

    row_stride = lax.iota(jnp.int32, LANES) * FIELDS
    row_ids0 = lax.iota(jnp.int32, LANES)
    zeros16 = jnp.zeros((LANES,), jnp.int32)
    bias_s = bias_v[...][0]

    def chunk_body(c, carry):
        ids0 = c * (LANES * FIELDS) + row_stride
        acc = jnp.zeros((LANES,), jnp.float32)
        for f in range(FIELDS):
            ids = ids0 + f
            e = plsc.load_gather(emb_v, [ids])
            v = plsc.load_gather(val_v, [ids])
            acc = acc + e * v
        out_v[pl.ds(c * LANES, LANES)] = acc + bias_s
        return carry

    lax.fori_loop(0, BPW // LANES, chunk_body, 0)

    pltpu.sync_copy(out_v, out_hbm.at[pl.ds(wid * BPW, BPW)])


def kernel(idx, val, fc_weight, bias):
    out = _fl_kernel(idx.reshape(-1), val.reshape(-1),
                     lax.squeeze(fc_weight, (1,)), bias)
    return out.reshape(BATCH, 1)
